# baseline (speedup 1.0000x reference)
import jax
import jax.numpy as jnp
from jax import lax
from jax.experimental import pallas as pl
from jax.experimental.pallas import tpu as pltpu
from jax.experimental.pallas import tpu_sc as plsc

N = 10000
E = 320000
D = 128
NC = 2
NS = 16
NW = NC * NS
EPW = E // NW
CH = 80
NCH = EPW // CH
RPS = N // NS

_f32 = jnp.float32
_i32 = jnp.int32

_SC_PARAMS = pltpu.CompilerParams(
    use_tc_tiling_on_sc=False, needs_layout_passes=False)


def _branch(x, w_ref, b_ref, wfc_ref, k):
    h = jnp.dot(x, w_ref[...], preferred_element_type=_f32)
    h = jnp.maximum(h + b_ref[...], 0.0)
    f = wfc_ref[pl.ds(64 * k, 64), :]
    return jnp.dot(h, f, preferred_element_type=_f32)


def _dense3_body(x_ref, w3_ref, b3_ref, wfc_ref, p3_ref):
    p3_ref[...] = _branch(x_ref[...], w3_ref, b3_ref, wfc_ref, 2)


def _dense12_body(x_ref, w1_ref, b1_ref, w2_ref, b2_ref, wfc_ref,
                  p1_ref, p2_ref):
    x = x_ref[...]
    p1_ref[...] = _branch(x, w1_ref, b1_ref, wfc_ref, 0)
    p2_ref[...] = _branch(x, w2_ref, b2_ref, wfc_ref, 1)


_BLK = 2000
_XSPEC = pl.BlockSpec((_BLK, D), lambda i: (i, 0))
_WSPEC = pl.BlockSpec((D, 64), lambda i: (0, 0))
_BSPEC = pl.BlockSpec((1, 64), lambda i: (0, 0))
_FSPEC = pl.BlockSpec((192, 16), lambda i: (0, 0))
_OSPEC = pl.BlockSpec((_BLK, 16), lambda i: (i, 0))
_OSHAPE = jax.ShapeDtypeStruct((N, 16), _f32)


def _dense3_stage(x, w3, b3, wfc):
    return pl.pallas_call(
        _dense3_body,
        grid=(N // _BLK,),
        in_specs=[_XSPEC, _WSPEC, _BSPEC, _FSPEC],
        out_specs=_OSPEC,
        out_shape=_OSHAPE,
    )(x, w3, b3, wfc)


def _dense12_stage(x, w1, b1, w2, b2, wfc):
    return pl.pallas_call(
        _dense12_body,
        grid=(N // _BLK,),
        in_specs=[_XSPEC, _WSPEC, _BSPEC, _WSPEC, _BSPEC, _FSPEC],
        out_specs=[_OSPEC, _OSPEC],
        out_shape=[_OSHAPE, _OSHAPE],
    )(x, w1, b1, w2, b2, wfc)


def _zero_rows(buf, nrows):
    def z(r, carry):
        buf[r, :] = jnp.zeros((16,), _f32)
        return carry

    lax.fori_loop(0, nrows, z, 0)


def _stage_edges(rows_hbm, cols_hbm, vals_hbm, wid, rowv, colv, valv, s0, s1, s2):
    base = wid * EPW
    d0 = pltpu.make_async_copy(rows_hbm.at[pl.ds(base, EPW)], rowv, s0)
    d1 = pltpu.make_async_copy(cols_hbm.at[pl.ds(base, EPW)], colv, s1)
    d2 = pltpu.make_async_copy(vals_hbm.at[pl.ds(base, EPW)], valv, s2)
    d0.start()
    d1.start()
    d2.start()
    d0.wait()
    d1.wait()
    d2.wait()


def _edge_loop(tab_hbm, acc, rowv, colv, valv, g0, g1, sq0, sq1,
               gsem0, gsem1, ssem0, ssem1):
    def scale(j, g, sq):
        for i2 in range(CH // 16):
            vv = valv[pl.ds(j * CH + i2 * 16, 16)]
            for e in range(16):
                i = i2 * 16 + e
                vi = vv.at[jnp.full((16,), e, _i32)].get(
                    mode="promise_in_bounds")
                sq[i, :] = g[i, :] * vi

    def wait_gather(j, g, gsem):
        pltpu.make_async_copy(
            tab_hbm.at[colv.at[pl.ds(j * CH, CH)]], g, gsem).wait()

    def start_gather(j, g, gsem):
        pltpu.async_copy(tab_hbm.at[colv.at[pl.ds(j * CH, CH)]], g, gsem)

    def start_scatter(j, sq, ssem):
        pltpu.async_copy(sq, acc.at[rowv.at[pl.ds(j * CH, CH)]], ssem,
                         add=True)

    def wait_scatter(sq, ssem):
        pltpu.make_async_copy(sq, acc.at[rowv.at[pl.ds(0, CH)]], ssem).wait()

    def process(j, g, sq, gsem, ssem, first):
        wait_gather(j, g, gsem)
        if not first:
            wait_scatter(sq, ssem)
        scale(j, g, sq)
        start_scatter(j, sq, ssem)

    start_gather(0, g0, gsem0)
    start_gather(1, g1, gsem1)
    process(0, g0, sq0, gsem0, ssem0, True)
    start_gather(2, g0, gsem0)
    process(1, g1, sq1, gsem1, ssem1, True)
    start_gather(3, g1, gsem1)

    def pair(k, carry):
        j0 = 2 * k
        process(j0, g0, sq0, gsem0, ssem0, False)
        start_gather(j0 + 2, g0, gsem0)
        process(j0 + 1, g1, sq1, gsem1, ssem1, False)
        start_gather(j0 + 3, g1, gsem1)
        return carry

    lax.fori_loop(1, (NCH - 3) // 2, pair, 0)
    process(NCH - 3, g0, sq0, gsem0, ssem0, False)
    start_gather(NCH - 1, g0, gsem0)
    process(NCH - 2, g1, sq1, gsem1, ssem1, False)
    process(NCH - 1, g0, sq0, gsem0, ssem0, False)
    wait_scatter(sq0, ssem0)
    wait_scatter(sq1, ssem1)


def _spmm_a_body(rows_hbm, cols_hbm, vals_hbm, tab_hbm, out_hbm,
                 acc, rowv, colv, valv, g0, g1, sq0, sq1, zb,
                 gsem0, gsem1, ssem0, ssem1):
    c = lax.axis_index("c")
    s = lax.axis_index("s")
    sl = pl.ds(s * RPS, RPS)
    _zero_rows(zb, RPS)
    pltpu.sync_copy(zb, acc.at[sl])
    _stage_edges(rows_hbm, cols_hbm, vals_hbm, c * NS + s, rowv, colv,
                 valv, gsem0, gsem1, ssem0)
    plsc.subcore_barrier()
    _edge_loop(tab_hbm, acc, rowv, colv, valv, g0, g1, sq0, sq1,
               gsem0, gsem1, ssem0, ssem1)
    plsc.subcore_barrier()
    pltpu.sync_copy(acc.at[sl], out_hbm.at[c, sl])


def _spmm_b_body(rows_hbm, cols_hbm, vals_hbm, p2_hbm, pa_hbm,
                 out_hbm, u_hbm,
                 acc, rowv, colv, valv, g0, g1, sq0, sq1, ub, t0b, t1b,
                 gsem0, gsem1, ssem0, ssem1):
    c = lax.axis_index("c")
    s = lax.axis_index("s")
    sl = pl.ds(s * RPS, RPS)
    base = (c * NS + s) * EPW
    e0 = pltpu.make_async_copy(rows_hbm.at[pl.ds(base, EPW)], rowv, ssem1)
    e1 = pltpu.make_async_copy(cols_hbm.at[pl.ds(base, EPW)], colv, gsem1)
    e2 = pltpu.make_async_copy(vals_hbm.at[pl.ds(base, EPW)], valv, ssem0)
    e0.start()
    e1.start()
    e2.start()
    pltpu.sync_copy(p2_hbm.at[sl], ub)
    pltpu.sync_copy(pa_hbm.at[0, sl], t0b)
    pltpu.sync_copy(pa_hbm.at[1, sl], t1b)

    def add_row(r, carry):
        ub[r, :] = ub[r, :] + t0b[r, :] + t1b[r, :]
        return carry

    lax.fori_loop(0, RPS, add_row, 0)
    pltpu.sync_copy(ub, u_hbm.at[sl])
    _zero_rows(t0b, RPS)
    pltpu.sync_copy(t0b, acc.at[sl])
    e0.wait()
    e1.wait()
    e2.wait()
    plsc.subcore_barrier()
    _edge_loop(u_hbm, acc, rowv, colv, valv, g0, g1, sq0, sq1,
               gsem0, gsem1, ssem0, ssem1)
    plsc.subcore_barrier()
    pltpu.sync_copy(acc.at[sl], out_hbm.at[c, sl])


_BASE_SCRATCH = [
    pltpu.VMEM_SHARED((N, 16), _f32),
    pltpu.VMEM((EPW,), _i32),
    pltpu.VMEM((EPW,), _i32),
    pltpu.VMEM((EPW,), _f32),
    pltpu.VMEM((CH, 16), _f32),
    pltpu.VMEM((CH, 16), _f32),
    pltpu.VMEM((CH, 16), _f32),
    pltpu.VMEM((CH, 16), _f32),
]
_SEMS = [pltpu.SemaphoreType.DMA] * 4


def _spmm_a_stage(rows, cols, vals, tab):
    mesh = plsc.VectorSubcoreMesh(core_axis_name="c", subcore_axis_name="s")
    f = pl.kernel(
        _spmm_a_body,
        out_type=jax.ShapeDtypeStruct((NC, N, 16), _f32),
        mesh=mesh,
        compiler_params=_SC_PARAMS,
        scratch_types=_BASE_SCRATCH + [pltpu.VMEM((RPS, 16), _f32)] + _SEMS,
    )
    return f(rows, cols, vals, tab)


def _spmm_b_stage(rows, cols, vals, p2, pa):
    mesh = plsc.VectorSubcoreMesh(core_axis_name="c", subcore_axis_name="s")
    f = pl.kernel(
        _spmm_b_body,
        out_type=[
            jax.ShapeDtypeStruct((NC, N, 16), _f32),
            jax.ShapeDtypeStruct((N, 16), _f32),
        ],
        mesh=mesh,
        compiler_params=_SC_PARAMS,
        scratch_types=_BASE_SCRATCH + [
            pltpu.VMEM((RPS, 16), _f32),
            pltpu.VMEM((RPS, 16), _f32),
            pltpu.VMEM((RPS, 16), _f32),
        ] + _SEMS,
    )
    return f(rows, cols, vals, p2, pa)


def _final_body(p1_ref, qb_ref, bfc_ref, out_ref):
    logits = p1_ref[...] + qb_ref[0] + qb_ref[1] + bfc_ref[...]
    m = jnp.max(logits, axis=1, keepdims=True)
    sh = logits - m
    lse = jnp.log(jnp.sum(jnp.exp(sh), axis=1, keepdims=True))
    out_ref[...] = sh - lse


def _final_stage(p1, outb, bfc):
    return pl.pallas_call(
        _final_body,
        grid=(N // _BLK,),
        in_specs=[
            _OSPEC,
            pl.BlockSpec((NC, _BLK, 16), lambda i: (0, i, 0)),
            pl.BlockSpec((1, 16), lambda i: (0, 0)),
        ],
        out_specs=_OSPEC,
        out_shape=_OSHAPE,
    )(p1, outb, bfc)


def kernel(adj_indices, adj_values, features, W1, b1, W2, b2, W3, b3,
           W_fc, b_fc):
    bfc = b_fc.reshape(1, 16)
    rows = adj_indices[0]
    cols = adj_indices[1]
    p3 = _dense3_stage(features, W3, b3.reshape(1, 64), W_fc)
    pa = _spmm_a_stage(rows, cols, adj_values, p3)
    p1, p2 = _dense12_stage(features, W1, b1.reshape(1, 64),
                            W2, b2.reshape(1, 64), W_fc)
    pb, _ = _spmm_b_stage(rows, cols, adj_values, p2, pa)
    return _final_stage(p1, pb, bfc)


# speedup vs baseline: 1.0539x; 1.0539x over previous
import jax
import jax.numpy as jnp
from jax import lax
from jax.experimental import pallas as pl
from jax.experimental.pallas import tpu as pltpu
from jax.experimental.pallas import tpu_sc as plsc

N = 10000
E = 320000
D = 128
NC = 2
NS = 16
NW = NC * NS
EPW = E // NW
CH = 80
NCH = EPW // CH
RPS = N // NS

_f32 = jnp.float32
_i32 = jnp.int32

_SC_PARAMS = pltpu.CompilerParams(
    use_tc_tiling_on_sc=False, needs_layout_passes=False)


def _branch(x, w_ref, b_ref, wfc_ref, k):
    h = jnp.dot(x, w_ref[...], preferred_element_type=_f32)
    h = jnp.maximum(h + b_ref[...], 0.0)
    f = wfc_ref[pl.ds(64 * k, 64), :]
    return jnp.dot(h, f, preferred_element_type=_f32)


def _dense3_body(x_ref, w3_ref, b3_ref, wfc_ref, p3_ref):
    p3_ref[...] = _branch(x_ref[...], w3_ref, b3_ref, wfc_ref, 2)


def _dense12_body(x_ref, w1_ref, b1_ref, w2_ref, b2_ref, wfc_ref,
                  p1_ref, p2_ref):
    x = x_ref[...]
    p1_ref[...] = _branch(x, w1_ref, b1_ref, wfc_ref, 0)
    p2_ref[...] = _branch(x, w2_ref, b2_ref, wfc_ref, 1)


_BLK = 2000
_XSPEC = pl.BlockSpec((_BLK, D), lambda i: (i, 0))
_WSPEC = pl.BlockSpec((D, 64), lambda i: (0, 0))
_BSPEC = pl.BlockSpec((1, 64), lambda i: (0, 0))
_FSPEC = pl.BlockSpec((192, 16), lambda i: (0, 0))
_OSPEC = pl.BlockSpec((_BLK, 16), lambda i: (i, 0))
_OSHAPE = jax.ShapeDtypeStruct((N, 16), _f32)


def _dense3_stage(x, w3, b3, wfc):
    return pl.pallas_call(
        _dense3_body,
        grid=(N // _BLK,),
        in_specs=[_XSPEC, _WSPEC, _BSPEC, _FSPEC],
        out_specs=_OSPEC,
        out_shape=_OSHAPE,
    )(x, w3, b3, wfc)


def _dense12_stage(x, w1, b1, w2, b2, wfc):
    return pl.pallas_call(
        _dense12_body,
        grid=(N // _BLK,),
        in_specs=[_XSPEC, _WSPEC, _BSPEC, _WSPEC, _BSPEC, _FSPEC],
        out_specs=[_OSPEC, _OSPEC],
        out_shape=[_OSHAPE, _OSHAPE],
    )(x, w1, b1, w2, b2, wfc)


def _zero_rows(buf, nrows):
    def z(r, carry):
        buf[r, :] = jnp.zeros((16,), _f32)
        return carry

    lax.fori_loop(0, nrows, z, 0)


def _stage_edges(adj_hbm, vals_hbm, wid, rowv, colv, valv, s0, s1, s2):
    base = wid * EPW
    d0 = pltpu.make_async_copy(adj_hbm.at[0, pl.ds(base, EPW)], rowv, s0)
    d1 = pltpu.make_async_copy(adj_hbm.at[1, pl.ds(base, EPW)], colv, s1)
    d2 = pltpu.make_async_copy(vals_hbm.at[pl.ds(base, EPW)], valv, s2)
    d0.start()
    d1.start()
    d2.start()
    d0.wait()
    d1.wait()
    d2.wait()


def _edge_loop(tab_hbm, acc, rowv, colv, valv, g0, g1, sq0, sq1,
               gsem0, gsem1, ssem0, ssem1):
    def scale(j, g, sq):
        for i2 in range(CH // 16):
            vv = valv[pl.ds(j * CH + i2 * 16, 16)]
            for e in range(16):
                i = i2 * 16 + e
                vi = vv.at[jnp.full((16,), e, _i32)].get(
                    mode="promise_in_bounds")
                sq[i, :] = g[i, :] * vi

    def wait_gather(j, g, gsem):
        pltpu.make_async_copy(
            tab_hbm.at[colv.at[pl.ds(j * CH, CH)]], g, gsem).wait()

    def start_gather(j, g, gsem):
        pltpu.async_copy(tab_hbm.at[colv.at[pl.ds(j * CH, CH)]], g, gsem)

    def start_scatter(j, sq, ssem):
        pltpu.async_copy(sq, acc.at[rowv.at[pl.ds(j * CH, CH)]], ssem,
                         add=True)

    def wait_scatter(sq, ssem):
        pltpu.make_async_copy(sq, acc.at[rowv.at[pl.ds(0, CH)]], ssem).wait()

    def process(j, g, sq, gsem, ssem, first):
        wait_gather(j, g, gsem)
        if not first:
            wait_scatter(sq, ssem)
        scale(j, g, sq)
        start_scatter(j, sq, ssem)

    start_gather(0, g0, gsem0)
    start_gather(1, g1, gsem1)
    process(0, g0, sq0, gsem0, ssem0, True)
    start_gather(2, g0, gsem0)
    process(1, g1, sq1, gsem1, ssem1, True)
    start_gather(3, g1, gsem1)

    def pair(k, carry):
        j0 = 2 * k
        process(j0, g0, sq0, gsem0, ssem0, False)
        start_gather(j0 + 2, g0, gsem0)
        process(j0 + 1, g1, sq1, gsem1, ssem1, False)
        start_gather(j0 + 3, g1, gsem1)
        return carry

    lax.fori_loop(1, (NCH - 3) // 2, pair, 0)
    process(NCH - 3, g0, sq0, gsem0, ssem0, False)
    start_gather(NCH - 1, g0, gsem0)
    process(NCH - 2, g1, sq1, gsem1, ssem1, False)
    process(NCH - 1, g0, sq0, gsem0, ssem0, False)
    wait_scatter(sq0, ssem0)
    wait_scatter(sq1, ssem1)


def _spmm_a_body(adj_hbm, vals_hbm, tab_hbm, out_hbm,
                 acc, rowv, colv, valv, g0, g1, sq0, sq1, zb,
                 gsem0, gsem1, ssem0, ssem1):
    c = lax.axis_index("c")
    s = lax.axis_index("s")
    sl = pl.ds(s * RPS, RPS)
    _zero_rows(zb, RPS)
    pltpu.sync_copy(zb, acc.at[sl])
    _stage_edges(adj_hbm, vals_hbm, c * NS + s, rowv, colv, valv,
                 gsem0, gsem1, ssem0)
    plsc.subcore_barrier()
    _edge_loop(tab_hbm, acc, rowv, colv, valv, g0, g1, sq0, sq1,
               gsem0, gsem1, ssem0, ssem1)
    plsc.subcore_barrier()
    pltpu.sync_copy(acc.at[sl], out_hbm.at[c, sl])


def _spmm_b_body(adj_hbm, vals_hbm, p2_hbm, pa_hbm,
                 out_hbm, u_hbm,
                 acc, rowv, colv, valv, g0, g1, sq0, sq1, ub, t0b, t1b,
                 gsem0, gsem1, ssem0, ssem1):
    c = lax.axis_index("c")
    s = lax.axis_index("s")
    sl = pl.ds(s * RPS, RPS)
    base = (c * NS + s) * EPW
    e0 = pltpu.make_async_copy(adj_hbm.at[0, pl.ds(base, EPW)], rowv, ssem1)
    e1 = pltpu.make_async_copy(adj_hbm.at[1, pl.ds(base, EPW)], colv, gsem1)
    e2 = pltpu.make_async_copy(vals_hbm.at[pl.ds(base, EPW)], valv, ssem0)
    e0.start()
    e1.start()
    e2.start()
    pltpu.sync_copy(p2_hbm.at[sl], ub)
    pltpu.sync_copy(pa_hbm.at[0, sl], t0b)
    pltpu.sync_copy(pa_hbm.at[1, sl], t1b)

    def add_row(r, carry):
        ub[r, :] = ub[r, :] + t0b[r, :] + t1b[r, :]
        return carry

    lax.fori_loop(0, RPS, add_row, 0)
    pltpu.sync_copy(ub, u_hbm.at[sl])
    _zero_rows(t0b, RPS)
    pltpu.sync_copy(t0b, acc.at[sl])
    e0.wait()
    e1.wait()
    e2.wait()
    plsc.subcore_barrier()
    _edge_loop(u_hbm, acc, rowv, colv, valv, g0, g1, sq0, sq1,
               gsem0, gsem1, ssem0, ssem1)
    plsc.subcore_barrier()
    pltpu.sync_copy(acc.at[sl], out_hbm.at[c, sl])


_BASE_SCRATCH = [
    pltpu.VMEM_SHARED((N, 16), _f32),
    pltpu.VMEM((EPW,), _i32),
    pltpu.VMEM((EPW,), _i32),
    pltpu.VMEM((EPW,), _f32),
    pltpu.VMEM((CH, 16), _f32),
    pltpu.VMEM((CH, 16), _f32),
    pltpu.VMEM((CH, 16), _f32),
    pltpu.VMEM((CH, 16), _f32),
]
_SEMS = [pltpu.SemaphoreType.DMA] * 4


def _spmm_a_stage(adj, vals, tab):
    mesh = plsc.VectorSubcoreMesh(core_axis_name="c", subcore_axis_name="s")
    f = pl.kernel(
        _spmm_a_body,
        out_type=jax.ShapeDtypeStruct((NC, N, 16), _f32),
        mesh=mesh,
        compiler_params=_SC_PARAMS,
        scratch_types=_BASE_SCRATCH + [pltpu.VMEM((RPS, 16), _f32)] + _SEMS,
    )
    return f(adj, vals, tab)


def _spmm_b_stage(adj, vals, p2, pa):
    mesh = plsc.VectorSubcoreMesh(core_axis_name="c", subcore_axis_name="s")
    f = pl.kernel(
        _spmm_b_body,
        out_type=[
            jax.ShapeDtypeStruct((NC, N, 16), _f32),
            jax.ShapeDtypeStruct((N, 16), _f32),
        ],
        mesh=mesh,
        compiler_params=_SC_PARAMS,
        scratch_types=_BASE_SCRATCH + [
            pltpu.VMEM((RPS, 16), _f32),
            pltpu.VMEM((RPS, 16), _f32),
            pltpu.VMEM((RPS, 16), _f32),
        ] + _SEMS,
    )
    return f(adj, vals, p2, pa)


def _final_body(p1_ref, qb_ref, bfc_ref, out_ref):
    logits = p1_ref[...] + qb_ref[0] + qb_ref[1] + bfc_ref[...]
    m = jnp.max(logits, axis=1, keepdims=True)
    sh = logits - m
    lse = jnp.log(jnp.sum(jnp.exp(sh), axis=1, keepdims=True))
    out_ref[...] = sh - lse


def _final_stage(p1, outb, bfc):
    return pl.pallas_call(
        _final_body,
        grid=(N // _BLK,),
        in_specs=[
            _OSPEC,
            pl.BlockSpec((NC, _BLK, 16), lambda i: (0, i, 0)),
            pl.BlockSpec((1, 16), lambda i: (0, 0)),
        ],
        out_specs=_OSPEC,
        out_shape=_OSHAPE,
    )(p1, outb, bfc)


def kernel(adj_indices, adj_values, features, W1, b1, W2, b2, W3, b3,
           W_fc, b_fc):
    bfc = b_fc.reshape(1, 16)
    p3 = _dense3_stage(features, W3, b3.reshape(1, 64), W_fc)
    pa = _spmm_a_stage(adj_indices, adj_values, p3)
    p1, p2 = _dense12_stage(features, W1, b1.reshape(1, 64),
                            W2, b2.reshape(1, 64), W_fc)
    pb, _ = _spmm_b_stage(adj_indices, adj_values, p2, pa)
    return _final_stage(p1, pb, bfc)
